# baseline (speedup 1.0000x reference)
import functools

import jax
import jax.numpy as jnp
from jax import lax
from jax.experimental import pallas as pl
from jax.experimental.pallas import tpu as pltpu
from jax.experimental.pallas import tpu_sc as plsc

NODE = 128
EDGE = 16
GLB = 32
HID = 64
NN = 10000
NE = 320000
NG = 16

NC = 2
NS = 16
L = 16
NW = NC * NS
EPW = NE // NW
CH = 80
NCHUNK = EPW // CH
ACC_W = 80
NNP = 10240
ZROWS = NNP // NS
ZB = 128

SELU_ALPHA = 1.6732632423543772
SELU_SCALE = 1.0507009873554805

_f32 = jnp.float32


def _selu(v):
    return SELU_SCALE * jnp.where(v > 0, v, SELU_ALPHA * (jnp.exp(v) - 1.0))


def _x1_body(x_ref, w_ref, o_ref):
    o_ref[...] = jnp.dot(x_ref[...], w_ref[...],
                         preferred_element_type=_f32)


def _x1_call(x, w1x):
    blk = 2000
    return pl.pallas_call(
        _x1_body,
        out_shape=jax.ShapeDtypeStruct((NN, HID), _f32),
        grid=(NN // blk,),
        in_specs=[
            pl.BlockSpec((blk, NODE), lambda i: (i, 0)),
            pl.BlockSpec((NODE, HID), lambda i: (0, 0)),
        ],
        out_specs=pl.BlockSpec((blk, HID), lambda i: (i, 0)),
    )(x, w1x)


def _e1_body(ea_ref, w_ref, b_ref, o_ref):
    o_ref[...] = jnp.dot(ea_ref[...], w_ref[...],
                         preferred_element_type=_f32) + b_ref[...]


def _e1_call(edge_attr, w1e, b1):
    blk = 4000
    return pl.pallas_call(
        _e1_body,
        out_shape=jax.ShapeDtypeStruct((NE, HID), _f32),
        grid=(NE // blk,),
        in_specs=[
            pl.BlockSpec((blk, EDGE), lambda i: (i, 0)),
            pl.BlockSpec((EDGE, HID), lambda i: (0, 0)),
            pl.BlockSpec((1, HID), lambda i: (0, 0)),
        ],
        out_specs=pl.BlockSpec((blk, HID), lambda i: (i, 0)),
    )(edge_attr, w1e, b1)


def _sc_body(x1_hbm, e1_hbm, row2_hbm, col2_hbm, out_hbm,
             rowb, colb, xg0, xg1, ec0, ec1, vb0, vb1, zbuf, acc,
             gs0, gs1, es0, es1, ss0, ss1):
    cid = lax.axis_index("c")
    sid = lax.axis_index("s")
    wid = sid * NC + cid
    cbase = wid * NCHUNK

    zero16 = jnp.zeros((L,), _f32)

    def zb_loop(i, carry):
        for j in range(ACC_W // L):
            zbuf[i, pl.ds(j * L, L)] = zero16
        return carry

    lax.fori_loop(0, ZB, zb_loop, 0)
    for k in range(ZROWS // ZB):
        pltpu.sync_copy(zbuf, acc.at[pl.ds(sid * ZROWS + k * ZB, ZB)])

    cntvec = jnp.where(lax.iota(jnp.int32, L) == 0, _f32(1.0), _f32(0.0))

    def vb_loop(i, carry):
        vb0[i, pl.ds(HID, L)] = cntvec
        vb1[i, pl.ds(HID, L)] = cntvec
        return carry

    lax.fori_loop(0, CH, vb_loop, 0)

    pltpu.sync_copy(row2_hbm.at[pl.ds(cbase, NCHUNK)], rowb)
    pltpu.sync_copy(col2_hbm.at[pl.ds(cbase, NCHUNK)], colb)

    plsc.subcore_barrier()

    def issue(k, xgb, ecb, gsem, esem):
        pltpu.async_copy(x1_hbm.at[rowb.at[k]], xgb, gsem)
        ebase = pl.multiple_of((cbase + k) * CH, 16)
        pltpu.async_copy(e1_hbm.at[pl.ds(ebase, CH)], ecb, esem)

    def drain(xgb, ecb, gsem, esem):
        pltpu.make_async_copy(x1_hbm.at[rowb.at[0]], xgb, gsem).wait()
        pltpu.make_async_copy(e1_hbm.at[pl.ds(0, CH)], ecb, esem).wait()

    def wait_scatter(vbb, ssb):
        pltpu.make_async_copy(vbb, acc.at[colb.at[0]], ssb).wait()

    def compute(xgb, ecb, vbb):
        @plsc.parallel_loop(0, CH, 1, unroll=4)
        def _(i):
            for j in range(HID // L):
                v = xgb[i, pl.ds(j * L, L)] + ecb[i, pl.ds(j * L, L)]
                vbb[i, pl.ds(j * L, L)] = _selu(v)

    issue(0, xg0, ec0, gs0, es0)
    issue(1, xg1, ec1, gs1, es1)

    def pair(p, carry):
        k0 = 2 * p

        @pl.when(p > 0)
        def _():
            wait_scatter(vb0, ss0)
        drain(xg0, ec0, gs0, es0)
        compute(xg0, ec0, vb0)
        pltpu.async_copy(vb0, acc.at[colb.at[k0]], ss0, add=True)
        issue(k0 + 2, xg0, ec0, gs0, es0)

        @pl.when(p > 0)
        def _():
            wait_scatter(vb1, ss1)
        drain(xg1, ec1, gs1, es1)
        compute(xg1, ec1, vb1)
        pltpu.async_copy(vb1, acc.at[colb.at[k0 + 1]], ss1, add=True)

        @pl.when(p < (NCHUNK - 3) // 2)
        def _():
            issue(k0 + 3, xg1, ec1, gs1, es1)
        return carry

    lax.fori_loop(0, (NCHUNK - 1) // 2, pair, 0)

    wait_scatter(vb0, ss0)
    drain(xg0, ec0, gs0, es0)
    compute(xg0, ec0, vb0)
    pltpu.sync_copy(vb0, acc.at[colb.at[NCHUNK - 1]], add=True)
    wait_scatter(vb1, ss1)

    plsc.subcore_barrier()
    pltpu.sync_copy(acc.at[pl.ds(sid * ZROWS, ZROWS)],
                    out_hbm.at[cid, pl.ds(sid * ZROWS, ZROWS)])


def _sc_call(x1, e1, row2, col2):
    mesh = plsc.VectorSubcoreMesh(core_axis_name="c", subcore_axis_name="s",
                                  num_cores=NC, num_subcores=NS)
    f = functools.partial(
        pl.kernel,
        out_type=jax.ShapeDtypeStruct((NC, NNP, ACC_W), _f32),
        mesh=mesh,
        scratch_types=[
            pltpu.VMEM((NCHUNK, CH), jnp.int32),
            pltpu.VMEM((NCHUNK, CH), jnp.int32),
            pltpu.VMEM((CH, HID), _f32),
            pltpu.VMEM((CH, HID), _f32),
            pltpu.VMEM((CH, HID), _f32),
            pltpu.VMEM((CH, HID), _f32),
            pltpu.VMEM((CH, ACC_W), _f32),
            pltpu.VMEM((CH, ACC_W), _f32),
            pltpu.VMEM((ZB, ACC_W), _f32),
            pltpu.VMEM_SHARED((NNP, ACC_W), _f32),
            pltpu.SemaphoreType.DMA,
            pltpu.SemaphoreType.DMA,
            pltpu.SemaphoreType.DMA,
            pltpu.SemaphoreType.DMA,
            pltpu.SemaphoreType.DMA,
            pltpu.SemaphoreType.DMA,
        ],
        compiler_params=pltpu.CompilerParams(use_tc_tiling_on_sc=False),
    )(_sc_body)
    return f(x1, e1, row2, col2)


def _node_body(x_ref, p0_ref, p1_ref, bt_ref, u_ref,
               w2_ref, b2_ref, w3_ref, b3_ref, w4_ref, b4_ref, o_ref):
    s = p0_ref[:, 0:HID] + p1_ref[:, 0:HID]
    cnt = p0_ref[:, HID:HID + 1] + p1_ref[:, HID:HID + 1]
    sfull = jnp.dot(s, w2_ref[...], preferred_element_type=_f32) \
        + cnt * b2_ref[...]
    mean = sfull / jnp.maximum(cnt, 1.0)

    bt = bt_ref[0, 0, :]
    blk = bt.shape[0]
    onehot = (bt.reshape(blk, 1)
              == lax.broadcasted_iota(jnp.int32, (blk, NG), 1)).astype(_f32)
    uw = jnp.dot(u_ref[...], w3_ref[NODE + HID:, :],
                 preferred_element_type=_f32)

    h = (jnp.dot(x_ref[...], w3_ref[0:NODE, :],
                 preferred_element_type=_f32)
         + jnp.dot(mean, w3_ref[NODE:NODE + HID, :],
                   preferred_element_type=_f32)
         + jnp.dot(onehot, uw, preferred_element_type=_f32)
         + b3_ref[...])
    h = _selu(h)
    o_ref[...] = jnp.dot(h, w4_ref[...],
                         preferred_element_type=_f32) + b4_ref[...]


def _node_call(x, p0, p1, batch3, u, w2, b2, w3, b3, w4, b4):
    blk = 2000
    nb = NN // blk
    in2 = NODE + HID + GLB
    return pl.pallas_call(
        _node_body,
        out_shape=jax.ShapeDtypeStruct((NN, NODE), _f32),
        grid=(nb,),
        in_specs=[
            pl.BlockSpec((blk, NODE), lambda i: (i, 0)),
            pl.BlockSpec((blk, ACC_W), lambda i: (i, 0)),
            pl.BlockSpec((blk, ACC_W), lambda i: (i, 0)),
            pl.BlockSpec((1, 1, blk), lambda i: (i, 0, 0)),
            pl.BlockSpec((NG, GLB), lambda i: (0, 0)),
            pl.BlockSpec((HID, HID), lambda i: (0, 0)),
            pl.BlockSpec((1, HID), lambda i: (0, 0)),
            pl.BlockSpec((in2, HID), lambda i: (0, 0)),
            pl.BlockSpec((1, HID), lambda i: (0, 0)),
            pl.BlockSpec((HID, NODE), lambda i: (0, 0)),
            pl.BlockSpec((1, NODE), lambda i: (0, 0)),
        ],
        out_specs=pl.BlockSpec((blk, NODE), lambda i: (i, 0)),
    )(x, p0, p1, batch3, u, w2, b2, w3, b3, w4, b4)


def kernel(x, edge_index, edge_attr, u, batch, W1, b1, W2, b2, W3, b3, W4, b4):
    row = edge_index[0].astype(jnp.int32)
    col = edge_index[1].astype(jnp.int32)

    x1 = _x1_call(x, W1[:NODE])
    e1 = _e1_call(edge_attr, W1[NODE:], b1.reshape(1, HID))
    parts = _sc_call(x1, e1, row.reshape(NE // CH, CH), col.reshape(NE // CH, CH))

    batch3 = batch.astype(jnp.int32).reshape(NN // 2000, 1, 2000)
    return _node_call(x, parts[0], parts[1], batch3, u,
                      W2, b2.reshape(1, HID), W3, b3.reshape(1, HID),
                      W4, b4.reshape(1, NODE))


# speedup vs baseline: 1.8932x; 1.8932x over previous
import functools

import jax
import jax.numpy as jnp
from jax import lax
from jax.experimental import pallas as pl
from jax.experimental.pallas import tpu as pltpu
from jax.experimental.pallas import tpu_sc as plsc

NODE = 128
EDGE = 16
GLB = 32
HID = 64
NN = 10000
NE = 320000
NG = 16

NC = 2
NS = 16
L = 16
NW = NC * NS
EPW = NE // NW
CH = 80
NCHUNK = EPW // CH
ACC_W = 80
NNP = 10240
ZROWS = NNP // NS
ZB = 128

SELU_ALPHA = 1.6732632423543772
SELU_SCALE = 1.0507009873554805

_f32 = jnp.float32


def _selu(v):
    return SELU_SCALE * jnp.where(v > 0, v, SELU_ALPHA * (jnp.exp(v) - 1.0))


def _x1_body(x_ref, w_ref, o_ref):
    o_ref[...] = jnp.dot(x_ref[...], w_ref[...],
                         preferred_element_type=_f32)


def _x1_call(x, w1x):
    blk = 2000
    return pl.pallas_call(
        _x1_body,
        out_shape=jax.ShapeDtypeStruct((NN, HID), _f32),
        grid=(NN // blk,),
        in_specs=[
            pl.BlockSpec((blk, NODE), lambda i: (i, 0)),
            pl.BlockSpec((NODE, HID), lambda i: (0, 0)),
        ],
        out_specs=pl.BlockSpec((blk, HID), lambda i: (i, 0)),
    )(x, w1x)


def _e1_body(eat_ref, w_ref, b_ref, o_ref):
    blk = eat_ref.shape[1]
    acc = lax.dot_general(eat_ref[...], w_ref[...],
                          (((0,), (0,)), ((), ())),
                          preferred_element_type=_f32) + b_ref[...]
    padded = jnp.concatenate([acc, jnp.zeros((blk, HID), _f32)], axis=1)
    o_ref[...] = padded.reshape(blk // 8, 8, 2 * HID)


def _e1_call(eat, w1e, b1):
    blk = 6400
    return pl.pallas_call(
        _e1_body,
        out_shape=jax.ShapeDtypeStruct((NE // 8, 8, 2 * HID), _f32),
        grid=(NE // blk,),
        in_specs=[
            pl.BlockSpec((EDGE, blk), lambda i: (0, i)),
            pl.BlockSpec((EDGE, HID), lambda i: (0, 0)),
            pl.BlockSpec((1, HID), lambda i: (0, 0)),
        ],
        out_specs=pl.BlockSpec((blk // 8, 8, 2 * HID), lambda i: (i, 0, 0)),
    )(eat, w1e, b1)


def _sc_body(x1_hbm, e1_hbm, row_hbm, col_hbm, out0_hbm, out1_hbm,
             rowb, colb, xg0, xg1, ec0, ec1, vb0, vb1, zbuf, acc,
             gs0, gs1, es0, es1, ss0, ss1):
    cid = lax.axis_index("c")
    sid = lax.axis_index("s")
    wid = sid * NC + cid
    cbase = wid * NCHUNK

    zero16 = jnp.zeros((L,), _f32)

    def zb_loop(i, carry):
        for j in range(ACC_W // L):
            zbuf[i, pl.ds(j * L, L)] = zero16
        return carry

    lax.fori_loop(0, ZB, zb_loop, 0)
    for k in range(ZROWS // ZB):
        pltpu.sync_copy(zbuf, acc.at[pl.ds(sid * ZROWS + k * ZB, ZB)])

    cntvec = jnp.where(lax.iota(jnp.int32, L) == 0, _f32(1.0), _f32(0.0))

    def vb_loop(i, carry):
        vb0[i, pl.ds(HID, L)] = cntvec
        vb1[i, pl.ds(HID, L)] = cntvec
        return carry

    lax.fori_loop(0, CH, vb_loop, 0)

    ebase0 = pl.multiple_of(wid * EPW, 16)
    pltpu.sync_copy(row_hbm.at[pl.ds(ebase0, EPW)], rowb)
    pltpu.sync_copy(col_hbm.at[pl.ds(ebase0, EPW)], colb)

    plsc.subcore_barrier()

    def issue(k, xgb, ecb, gsem, esem):
        pltpu.async_copy(x1_hbm.at[rowb.at[pl.ds(pl.multiple_of(k * CH, 16), CH)]], xgb, gsem)
        ebase = pl.multiple_of((cbase + k) * CH * 2 * HID, 1024)
        pltpu.async_copy(e1_hbm.at[pl.ds(ebase, CH * 2 * HID)], ecb, esem)

    def drain(xgb, ecb, gsem, esem):
        pltpu.make_async_copy(x1_hbm.at[rowb.at[pl.ds(0, CH)]], xgb, gsem).wait()
        pltpu.make_async_copy(e1_hbm.at[pl.ds(0, CH * 2 * HID)], ecb, esem).wait()

    def wait_scatter(vbb, ssb):
        pltpu.make_async_copy(vbb, acc.at[colb.at[pl.ds(0, CH)]], ssb).wait()

    def compute(xgb, ecb, vbb):
        @plsc.parallel_loop(0, CH, 1, unroll=4)
        def _(i):
            for j in range(HID // L):
                v = xgb[i, pl.ds(j * L, L)] + ecb[pl.ds(i * 2 * HID + j * L, L)]
                vbb[i, pl.ds(j * L, L)] = _selu(v)

    issue(0, xg0, ec0, gs0, es0)
    issue(1, xg1, ec1, gs1, es1)

    def pair(p, carry):
        k0 = 2 * p

        @pl.when(p > 0)
        def _():
            wait_scatter(vb0, ss0)
        drain(xg0, ec0, gs0, es0)
        compute(xg0, ec0, vb0)
        pltpu.async_copy(vb0, acc.at[colb.at[pl.ds(pl.multiple_of(k0 * CH, 16), CH)]], ss0, add=True)
        issue(k0 + 2, xg0, ec0, gs0, es0)

        @pl.when(p > 0)
        def _():
            wait_scatter(vb1, ss1)
        drain(xg1, ec1, gs1, es1)
        compute(xg1, ec1, vb1)
        pltpu.async_copy(vb1, acc.at[colb.at[pl.ds(pl.multiple_of((k0 + 1) * CH, 16), CH)]], ss1, add=True)

        @pl.when(p < (NCHUNK - 3) // 2)
        def _():
            issue(k0 + 3, xg1, ec1, gs1, es1)
        return carry

    lax.fori_loop(0, (NCHUNK - 1) // 2, pair, 0)

    wait_scatter(vb0, ss0)
    drain(xg0, ec0, gs0, es0)
    compute(xg0, ec0, vb0)
    pltpu.sync_copy(vb0, acc.at[colb.at[pl.ds((NCHUNK - 1) * CH, CH)]], add=True)
    wait_scatter(vb1, ss1)

    plsc.subcore_barrier()

    @pl.when(cid == 0)
    def _():
        pltpu.sync_copy(acc.at[pl.ds(sid * ZROWS, ZROWS)],
                        out0_hbm.at[pl.ds(sid * ZROWS, ZROWS), pl.ds(0, ACC_W)])

    @pl.when(cid == 1)
    def _():
        pltpu.sync_copy(acc.at[pl.ds(sid * ZROWS, ZROWS)],
                        out1_hbm.at[pl.ds(sid * ZROWS, ZROWS), pl.ds(0, ACC_W)])


def _sc_call(x1, e1, row, col):
    mesh = plsc.VectorSubcoreMesh(core_axis_name="c", subcore_axis_name="s",
                                  num_cores=NC, num_subcores=NS)
    f = functools.partial(
        pl.kernel,
        out_type=(jax.ShapeDtypeStruct((NNP, 128), _f32),
                  jax.ShapeDtypeStruct((NNP, 128), _f32)),
        mesh=mesh,
        scratch_types=[
            pltpu.VMEM((EPW,), jnp.int32),
            pltpu.VMEM((EPW,), jnp.int32),
            pltpu.VMEM((CH, HID), _f32),
            pltpu.VMEM((CH, HID), _f32),
            pltpu.VMEM((CH * 2 * HID,), _f32),
            pltpu.VMEM((CH * 2 * HID,), _f32),
            pltpu.VMEM((CH, ACC_W), _f32),
            pltpu.VMEM((CH, ACC_W), _f32),
            pltpu.VMEM((ZB, ACC_W), _f32),
            pltpu.VMEM_SHARED((NNP, ACC_W), _f32),
            pltpu.SemaphoreType.DMA,
            pltpu.SemaphoreType.DMA,
            pltpu.SemaphoreType.DMA,
            pltpu.SemaphoreType.DMA,
            pltpu.SemaphoreType.DMA,
            pltpu.SemaphoreType.DMA,
        ],
        compiler_params=pltpu.CompilerParams(use_tc_tiling_on_sc=False),
    )(_sc_body)
    return f(x1, e1, row, col)


def _node_body(x_ref, p0_ref, p1_ref, bt_ref, u_ref,
               w2_ref, b2_ref, w3_ref, b3_ref, w4_ref, b4_ref, o_ref):
    s = p0_ref[:, 0:HID] + p1_ref[:, 0:HID]
    cnt = p0_ref[:, HID:HID + 1] + p1_ref[:, HID:HID + 1]
    sfull = jnp.dot(s, w2_ref[...], preferred_element_type=_f32) \
        + cnt * b2_ref[...]
    mean = sfull / jnp.maximum(cnt, 1.0)

    bt = bt_ref[0, 0, :]
    blk = bt.shape[0]
    onehot = (bt.reshape(blk, 1)
              == lax.broadcasted_iota(jnp.int32, (blk, NG), 1)).astype(_f32)
    uw = jnp.dot(u_ref[...], w3_ref[NODE + HID:, :],
                 preferred_element_type=_f32)

    h = (jnp.dot(x_ref[...], w3_ref[0:NODE, :],
                 preferred_element_type=_f32)
         + jnp.dot(mean, w3_ref[NODE:NODE + HID, :],
                   preferred_element_type=_f32)
         + jnp.dot(onehot, uw, preferred_element_type=_f32)
         + b3_ref[...])
    h = _selu(h)
    o_ref[...] = jnp.dot(h, w4_ref[...],
                         preferred_element_type=_f32) + b4_ref[...]


def _node_call(x, p0, p1, batch3, u, w2, b2, w3, b3, w4, b4):
    blk = 2000
    nb = NN // blk
    in2 = NODE + HID + GLB
    return pl.pallas_call(
        _node_body,
        out_shape=jax.ShapeDtypeStruct((NN, NODE), _f32),
        grid=(nb,),
        in_specs=[
            pl.BlockSpec((blk, NODE), lambda i: (i, 0)),
            pl.BlockSpec((blk, 128), lambda i: (i, 0)),
            pl.BlockSpec((blk, 128), lambda i: (i, 0)),
            pl.BlockSpec((1, 1, blk), lambda i: (i, 0, 0)),
            pl.BlockSpec((NG, GLB), lambda i: (0, 0)),
            pl.BlockSpec((HID, HID), lambda i: (0, 0)),
            pl.BlockSpec((1, HID), lambda i: (0, 0)),
            pl.BlockSpec((in2, HID), lambda i: (0, 0)),
            pl.BlockSpec((1, HID), lambda i: (0, 0)),
            pl.BlockSpec((HID, NODE), lambda i: (0, 0)),
            pl.BlockSpec((1, NODE), lambda i: (0, 0)),
        ],
        out_specs=pl.BlockSpec((blk, NODE), lambda i: (i, 0)),
    )(x, p0, p1, batch3, u, w2, b2, w3, b3, w4, b4)


def kernel(x, edge_index, edge_attr, u, batch, W1, b1, W2, b2, W3, b3, W4, b4):
    row = edge_index[0].astype(jnp.int32)
    col = edge_index[1].astype(jnp.int32)

    x1 = _x1_call(x, W1[:NODE])
    e1 = _e1_call(edge_attr.T, W1[NODE:], b1.reshape(1, HID))
    p0, p1 = _sc_call(x1, e1.reshape(NE * 2 * HID), row, col)

    batch3 = batch.astype(jnp.int32).reshape(NN // 2000, 1, 2000)
    return _node_call(x, p0, p1, batch3, u,
                      W2, b2.reshape(1, HID), W3, b3.reshape(1, HID),
                      W4, b4.reshape(1, NODE))
